# Initial kernel scaffold; baseline (speedup 1.0000x reference)
import jax
import jax.numpy as jnp
from jax.experimental import pallas as pl


def kernel(x, target):
    raise NotImplementedError("write your pallas kernel here")


# speedup vs baseline: 5.2834x; 5.2834x over previous
import math

import jax
import jax.numpy as jnp
from jax.experimental import pallas as pl
from jax.experimental.pallas import tpu as pltpu

_SMOOTHING = 0.1
_CONF = 1.0 - _SMOOTHING
_PAD = 0

_BR = 256
_BC = 3200


def _body(t_ref, x_ref, o_ref, *, bc, sval, kconst):
    i = pl.program_id(0)
    j = pl.program_id(1)

    @pl.when((i == 0) & (j == 0))
    def _init():
        o_ref[0, 0] = 0.0

    xb = x_ref[...]
    tgt = t_ref[...]
    br = xb.shape[0]
    colid = jax.lax.broadcasted_iota(jnp.int32, (br, bc), 1) + j * bc
    w = jnp.where(colid == tgt, -_CONF, -sval)
    w = jnp.where(colid == 0, 0.0, w)
    nonpad = tgt != _PAD
    w = jnp.where(nonpad, w, 0.0)
    part = jnp.sum(xb * w)

    @pl.when(j == 0)
    def _rowconst():
        cnt = jnp.sum(jnp.where(nonpad, 1.0, 0.0))
        o_ref[0, 0] += kconst * cnt

    o_ref[0, 0] += part


def kernel(x, target):
    n, v = x.shape
    sval = _SMOOTHING / (v - 2)
    kconst = (v - 2) * sval * math.log(sval) + _CONF * math.log(_CONF)
    tgt2d = target.astype(jnp.int32)[:, None]
    br, bc = _BR, _BC
    grid = (n // br, v // bc)
    import functools
    out = pl.pallas_call(
        functools.partial(_body, bc=bc, sval=sval, kconst=kconst),
        grid=grid,
        in_specs=[
            pl.BlockSpec((br, 1), lambda i, j: (i, 0)),
            pl.BlockSpec((br, bc), lambda i, j: (i, j)),
        ],
        out_specs=pl.BlockSpec(
            (1, 1), lambda i, j: (0, 0), memory_space=pltpu.SMEM
        ),
        out_shape=jax.ShapeDtypeStruct((1, 1), jnp.float32),
        compiler_params=pltpu.CompilerParams(
            dimension_semantics=("arbitrary", "arbitrary"),
        ),
    )(tgt2d, x)
    return out[0, 0]
